# speedup vs baseline: 2.5829x; 2.5829x over previous
import functools

import jax
import jax.numpy as jnp
from jax import lax
from jax.experimental import pallas as pl
from jax.experimental.pallas import tpu as pltpu
from jax.experimental.pallas import tpu_sc as plsc

B = 100000
D = 128
CHUNK = 128
NC = 2
NS = 16
NW = NC * NS
N_CHUNKS = (B + CHUNK - 1) // CHUNK
LAST_START = B - CHUNK

_mesh = plsc.VectorSubcoreMesh(core_axis_name="c", subcore_axis_name="s")


@functools.partial(
    pl.kernel,
    mesh=_mesh,
    out_type=jax.ShapeDtypeStruct((B, D), jnp.float32),
    scratch_types=[
        pltpu.VMEM((CHUNK,), jnp.int32),
        pltpu.VMEM((CHUNK, D), jnp.float32),
        pltpu.SemaphoreType.DMA,
    ],
)
def _gather_sc(table_hbm, idx_hbm, out_hbm, idx_v, rows_v, sem):
    wid = lax.axis_index("s") * NC + lax.axis_index("c")
    n_my = (N_CHUNKS - wid + NW - 1) // NW

    def body(i, carry):
        j = wid + i * NW
        start = pl.multiple_of(lax.min(j * CHUNK, LAST_START), 8)
        pltpu.sync_copy(idx_hbm.at[pl.ds(start, CHUNK)], idx_v)
        pltpu.async_copy(table_hbm.at[idx_v], rows_v, sem).wait()
        pltpu.sync_copy(rows_v, out_hbm.at[pl.ds(start, CHUNK)])
        return carry

    lax.fori_loop(0, n_my, body, 0)


def kernel(global_matrix, node_or_bond_graph_indices):
    idx = node_or_bond_graph_indices.astype(jnp.int32)
    return _gather_sc(global_matrix, idx)


# speedup vs baseline: 3.2895x; 1.2736x over previous
import functools

import jax
import jax.numpy as jnp
from jax import lax
from jax.experimental import pallas as pl
from jax.experimental.pallas import tpu as pltpu
from jax.experimental.pallas import tpu_sc as plsc

B = 100000
D = 128
CHUNK = 128
GC = 3
GROUP = GC * CHUNK
NC = 2
NS = 16
NW = NC * NS
N_G = (B + GROUP - 1) // GROUP
LAST_START = B - GROUP
MAX_G = (N_G + NW - 1) // NW
N_EXTRA = N_G - (MAX_G - 1) * NW
BLOCK = MAX_G * GROUP
IDX_LEN = ((MAX_G - 1) * (NW - 1) + N_EXTRA) * GROUP + BLOCK

_mesh = plsc.VectorSubcoreMesh(core_axis_name="c", subcore_axis_name="s")


@functools.partial(
    pl.kernel,
    mesh=_mesh,
    out_type=jax.ShapeDtypeStruct((B, D), jnp.float32),
    scratch_types=[
        pltpu.VMEM((BLOCK,), jnp.int32),
        pltpu.VMEM((2, GROUP, D), jnp.float32),
        pltpu.SemaphoreType.DMA,
        pltpu.SemaphoreType.DMA,
        pltpu.SemaphoreType.DMA,
        pltpu.SemaphoreType.DMA,
    ],
)
def _gather_sc(table_hbm, idx_hbm, out_hbm, idx_v, rows_v, sem_g0, sem_g1,
               sem_w0, sem_w1):
    sem_g = (sem_g0, sem_g1)
    sem_w = (sem_w0, sem_w1)
    wid = lax.axis_index("s") * NC + lax.axis_index("c")
    n_my = jnp.where(wid < N_EXTRA, MAX_G, MAX_G - 1)
    g0 = (MAX_G - 1) * wid + lax.min(wid, N_EXTRA)
    block_start = pl.multiple_of(g0 * GROUP, 8)

    pltpu.sync_copy(idx_hbm.at[pl.ds(block_start, BLOCK)], idx_v)

    def group_start(s):
        return pl.multiple_of(lax.min((g0 + s) * GROUP, LAST_START), 8)

    def fire_gathers(s, b):
        off = pl.multiple_of(group_start(s) - block_start, 8)
        for k in range(GC):
            pltpu.make_async_copy(
                table_hbm.at[idx_v.at[pl.ds(off + k * CHUNK, CHUNK)]],
                rows_v.at[b, pl.ds(k * CHUNK, CHUNK)],
                sem_g[b],
            ).start()

    def wait_gathers(s, b):
        off = pl.multiple_of(group_start(s) - block_start, 8)
        for k in range(GC):
            pltpu.make_async_copy(
                table_hbm.at[idx_v.at[pl.ds(off + k * CHUNK, CHUNK)]],
                rows_v.at[b, pl.ds(k * CHUNK, CHUNK)],
                sem_g[b],
            ).wait()

    def write_copy(s, b):
        return pltpu.make_async_copy(
            rows_v.at[b], out_hbm.at[pl.ds(group_start(s), GROUP)], sem_w[b])

    for s in range(MAX_G + 1):
        b = s % 2
        if s < MAX_G:
            @pl.when(s < n_my)
            def _(s=s, b=b):
                if s >= 2:
                    write_copy(s - 2, b).wait()
                fire_gathers(s, b)
        if s >= 1:
            sp, bp = s - 1, (s - 1) % 2
            @pl.when(sp < n_my)
            def _(sp=sp, bp=bp):
                wait_gathers(sp, bp)
                write_copy(sp, bp).start()

    for s in (MAX_G - 2, MAX_G - 1):
        @pl.when(s < n_my)
        def _(s=s):
            write_copy(s, s % 2).wait()


def kernel(global_matrix, node_or_bond_graph_indices):
    idx = node_or_bond_graph_indices.astype(jnp.int32)
    idx = jnp.pad(idx, (0, IDX_LEN - B))
    return _gather_sc(global_matrix, idx)


# speedup vs baseline: 3.3094x; 1.0060x over previous
import functools

import jax
import jax.numpy as jnp
from jax import lax
from jax.experimental import pallas as pl
from jax.experimental.pallas import tpu as pltpu
from jax.experimental.pallas import tpu_sc as plsc

B = 100000
D = 128
CHUNK = 128
GC = 2
GROUP = GC * CHUNK
NBUF = 3
NC = 2
NS = 16
NW = NC * NS
N_G = (B + GROUP - 1) // GROUP
LAST_START = B - GROUP
MAX_G = (N_G + NW - 1) // NW
N_EXTRA = N_G - (MAX_G - 1) * NW
BLOCK = MAX_G * GROUP
BLOCK_LAST = B - BLOCK

_mesh = plsc.VectorSubcoreMesh(core_axis_name="c", subcore_axis_name="s")


@functools.partial(
    pl.kernel,
    mesh=_mesh,
    out_type=jax.ShapeDtypeStruct((B, D), jnp.float32),
    scratch_types=[
        pltpu.VMEM((BLOCK,), jnp.int32),
        pltpu.VMEM((NBUF, GROUP, D), jnp.float32),
    ] + [pltpu.SemaphoreType.DMA] * (2 * NBUF),
)
def _gather_sc(table_hbm, idx_hbm, out_hbm, idx_v, rows_v, *sems):
    sem_g = sems[:NBUF]
    sem_w = sems[NBUF:]
    wid = lax.axis_index("s") * NC + lax.axis_index("c")
    n_my = jnp.where(wid < N_EXTRA, MAX_G, MAX_G - 1)
    g0 = (MAX_G - 1) * wid + lax.min(wid, N_EXTRA)
    block_start = pl.multiple_of(lax.min(g0 * GROUP, BLOCK_LAST), 8)

    pltpu.sync_copy(idx_hbm.at[pl.ds(block_start, BLOCK)], idx_v)

    def group_start(s):
        return pl.multiple_of(lax.min((g0 + s) * GROUP, LAST_START), 8)

    def gather_copy(s, b, k):
        off = pl.multiple_of(group_start(s) - block_start, 8)
        return pltpu.make_async_copy(
            table_hbm.at[idx_v.at[pl.ds(off + k * CHUNK, CHUNK)]],
            rows_v.at[b, pl.ds(k * CHUNK, CHUNK)],
            sem_g[b],
        )

    def write_copy(s, b):
        return pltpu.make_async_copy(
            rows_v.at[b], out_hbm.at[pl.ds(group_start(s), GROUP)], sem_w[b])

    for s in range(MAX_G + 1):
        b = s % NBUF
        if s < MAX_G:
            @pl.when(s < n_my)
            def _(s=s, b=b):
                if s >= NBUF:
                    write_copy(s - NBUF, b).wait()
                for k in range(GC):
                    gather_copy(s, b, k).start()
        if s >= 1:
            sp, bp = s - 1, (s - 1) % NBUF
            @pl.when(sp < n_my)
            def _(sp=sp, bp=bp):
                for k in range(GC):
                    gather_copy(sp, bp, k).wait()
                write_copy(sp, bp).start()

    for s in range(max(0, MAX_G - NBUF), MAX_G):
        @pl.when(s < n_my)
        def _(s=s):
            write_copy(s, s % NBUF).wait()


def kernel(global_matrix, node_or_bond_graph_indices):
    idx = node_or_bond_graph_indices.astype(jnp.int32)
    return _gather_sc(global_matrix, idx)


# speedup vs baseline: 5.1440x; 1.5544x over previous
import functools

import jax
import jax.numpy as jnp
from jax import lax
from jax.experimental import pallas as pl
from jax.experimental.pallas import tpu as pltpu
from jax.experimental.pallas import tpu_sc as plsc

B = 100000
D = 128
CHUNK = 128
GC = 2
GROUP = GC * CHUNK
NBUF = 3
NC = 2
NS = 16
NW = NC * NS
N_G = (B + GROUP - 1) // GROUP
LAST_START = B - GROUP
MAX_G = (N_G + NW - 1) // NW
N_EXTRA = N_G - (MAX_G - 1) * NW
BLOCK = MAX_G * GROUP
BLOCK_LAST = B - BLOCK

_mesh = plsc.VectorSubcoreMesh(core_axis_name="c", subcore_axis_name="s")


@functools.partial(
    pl.kernel,
    mesh=_mesh,
    out_type=jax.ShapeDtypeStruct((B, D), jnp.float32),
    scratch_types=[
        pltpu.VMEM((BLOCK,), jnp.int32),
        pltpu.VMEM((NBUF, GROUP, D), jnp.float32),
        pltpu.VMEM_SHARED((1024, D), jnp.float32),
    ] + [pltpu.SemaphoreType.DMA] * (2 * NBUF),
)
def _gather_sc(table_hbm, idx_hbm, out_hbm, idx_v, rows_v, table_sh, *sems):
    sem_g = sems[:NBUF]
    sem_w = sems[NBUF:]
    sub = lax.axis_index("s")
    wid = sub * NC + lax.axis_index("c")
    n_my = jnp.where(wid < N_EXTRA, MAX_G, MAX_G - 1)
    g0 = (MAX_G - 1) * wid + lax.min(wid, N_EXTRA)
    block_start = pl.multiple_of(lax.min(g0 * GROUP, BLOCK_LAST), 8)

    rows_per_sub = 1024 // NS
    tstart = pl.multiple_of(sub * rows_per_sub, 8)
    pltpu.sync_copy(table_hbm.at[pl.ds(tstart, rows_per_sub)],
                    table_sh.at[pl.ds(tstart, rows_per_sub)])
    pltpu.sync_copy(idx_hbm.at[pl.ds(block_start, BLOCK)], idx_v)
    plsc.subcore_barrier()

    def group_start(s):
        return pl.multiple_of(lax.min((g0 + s) * GROUP, LAST_START), 8)

    def gather_copy(s, b, k):
        off = pl.multiple_of(group_start(s) - block_start, 8)
        return pltpu.make_async_copy(
            table_sh.at[idx_v.at[pl.ds(off + k * CHUNK, CHUNK)]],
            rows_v.at[b, pl.ds(k * CHUNK, CHUNK)],
            sem_g[b],
        )

    def write_copy(s, b):
        return pltpu.make_async_copy(
            rows_v.at[b], out_hbm.at[pl.ds(group_start(s), GROUP)], sem_w[b])

    for s in range(MAX_G + 1):
        b = s % NBUF
        if s < MAX_G:
            @pl.when(s < n_my)
            def _(s=s, b=b):
                if s >= NBUF:
                    write_copy(s - NBUF, b).wait()
                for k in range(GC):
                    gather_copy(s, b, k).start()
        if s >= 1:
            sp, bp = s - 1, (s - 1) % NBUF
            @pl.when(sp < n_my)
            def _(sp=sp, bp=bp):
                for k in range(GC):
                    gather_copy(sp, bp, k).wait()
                write_copy(sp, bp).start()

    for s in range(max(0, MAX_G - NBUF), MAX_G):
        @pl.when(s < n_my)
        def _(s=s):
            write_copy(s, s % NBUF).wait()


def kernel(global_matrix, node_or_bond_graph_indices):
    idx = node_or_bond_graph_indices.astype(jnp.int32)
    return _gather_sc(global_matrix, idx)


# speedup vs baseline: 5.4573x; 1.0609x over previous
import functools

import jax
import jax.numpy as jnp
from jax import lax
from jax.experimental import pallas as pl
from jax.experimental.pallas import tpu as pltpu
from jax.experimental.pallas import tpu_sc as plsc

B = 100000
D = 128
CHUNK = 128
GC = 1
GROUP = GC * CHUNK
NBUF = 4
NC = 2
NS = 16
NW = NC * NS
N_G = (B + GROUP - 1) // GROUP
LAST_START = B - GROUP
MAX_G = (N_G + NW - 1) // NW
N_EXTRA = N_G - (MAX_G - 1) * NW
BLOCK = MAX_G * GROUP
BLOCK_LAST = B - BLOCK

_mesh = plsc.VectorSubcoreMesh(core_axis_name="c", subcore_axis_name="s")


@functools.partial(
    pl.kernel,
    mesh=_mesh,
    out_type=jax.ShapeDtypeStruct((B, D), jnp.float32),
    scratch_types=[
        pltpu.VMEM((BLOCK,), jnp.int32),
        pltpu.VMEM((NBUF, GROUP, D), jnp.float32),
        pltpu.VMEM_SHARED((1024, D), jnp.float32),
    ] + [pltpu.SemaphoreType.DMA] * (2 * NBUF),
)
def _gather_sc(table_hbm, idx_hbm, out_hbm, idx_v, rows_v, table_sh, *sems):
    sem_g = sems[:NBUF]
    sem_w = sems[NBUF:]
    sub = lax.axis_index("s")
    wid = sub * NC + lax.axis_index("c")
    n_my = jnp.where(wid < N_EXTRA, MAX_G, MAX_G - 1)
    g0 = (MAX_G - 1) * wid + lax.min(wid, N_EXTRA)
    block_start = pl.multiple_of(lax.min(g0 * GROUP, BLOCK_LAST), 8)

    rows_per_sub = 1024 // NS
    tstart = pl.multiple_of(sub * rows_per_sub, 8)
    pltpu.sync_copy(table_hbm.at[pl.ds(tstart, rows_per_sub)],
                    table_sh.at[pl.ds(tstart, rows_per_sub)])
    pltpu.sync_copy(idx_hbm.at[pl.ds(block_start, BLOCK)], idx_v)
    plsc.subcore_barrier()

    def group_start(s):
        return pl.multiple_of(lax.min((g0 + s) * GROUP, LAST_START), 8)

    def gather_copy(s, b, k):
        off = pl.multiple_of(group_start(s) - block_start, 8)
        return pltpu.make_async_copy(
            table_sh.at[idx_v.at[pl.ds(off + k * CHUNK, CHUNK)]],
            rows_v.at[b, pl.ds(k * CHUNK, CHUNK)],
            sem_g[b],
        )

    def write_copy(s, b):
        return pltpu.make_async_copy(
            rows_v.at[b], out_hbm.at[pl.ds(group_start(s), GROUP)], sem_w[b])

    for s in range(MAX_G + 1):
        b = s % NBUF
        if s < MAX_G:
            @pl.when(s < n_my)
            def _(s=s, b=b):
                if s >= NBUF:
                    write_copy(s - NBUF, b).wait()
                for k in range(GC):
                    gather_copy(s, b, k).start()
        if s >= 1:
            sp, bp = s - 1, (s - 1) % NBUF
            @pl.when(sp < n_my)
            def _(sp=sp, bp=bp):
                for k in range(GC):
                    gather_copy(sp, bp, k).wait()
                write_copy(sp, bp).start()

    for s in range(max(0, MAX_G - NBUF), MAX_G):
        @pl.when(s < n_my)
        def _(s=s):
            write_copy(s, s % NBUF).wait()


def kernel(global_matrix, node_or_bond_graph_indices):
    idx = node_or_bond_graph_indices.astype(jnp.int32)
    return _gather_sc(global_matrix, idx)


# speedup vs baseline: 5.4748x; 1.0032x over previous
import functools

import jax
import jax.numpy as jnp
from jax import lax
from jax.experimental import pallas as pl
from jax.experimental.pallas import tpu as pltpu
from jax.experimental.pallas import tpu_sc as plsc

B = 100000
D = 128
CHUNK = 128
GC = 1
GROUP = GC * CHUNK
NBUF = 6
NC = 2
NS = 16
NW = NC * NS
N_G = (B + GROUP - 1) // GROUP
LAST_START = B - GROUP
MAX_G = (N_G + NW - 1) // NW
N_EXTRA = N_G - (MAX_G - 1) * NW
BLOCK = MAX_G * GROUP
BLOCK_LAST = B - BLOCK

_mesh = plsc.VectorSubcoreMesh(core_axis_name="c", subcore_axis_name="s")


@functools.partial(
    pl.kernel,
    mesh=_mesh,
    out_type=jax.ShapeDtypeStruct((B, D), jnp.float32),
    scratch_types=[
        pltpu.VMEM((BLOCK,), jnp.int32),
        pltpu.VMEM((NBUF, GROUP, D), jnp.float32),
        pltpu.VMEM_SHARED((1024, D), jnp.float32),
    ] + [pltpu.SemaphoreType.DMA] * (2 * NBUF),
)
def _gather_sc(table_hbm, idx_hbm, out_hbm, idx_v, rows_v, table_sh, *sems):
    sem_g = sems[:NBUF]
    sem_w = sems[NBUF:]
    sub = lax.axis_index("s")
    wid = sub * NC + lax.axis_index("c")
    n_my = jnp.where(wid < N_EXTRA, MAX_G, MAX_G - 1)
    g0 = (MAX_G - 1) * wid + lax.min(wid, N_EXTRA)
    block_start = pl.multiple_of(lax.min(g0 * GROUP, BLOCK_LAST), 8)

    rows_per_sub = 1024 // NS
    tstart = pl.multiple_of(sub * rows_per_sub, 8)
    pltpu.sync_copy(table_hbm.at[pl.ds(tstart, rows_per_sub)],
                    table_sh.at[pl.ds(tstart, rows_per_sub)])
    pltpu.sync_copy(idx_hbm.at[pl.ds(block_start, BLOCK)], idx_v)
    plsc.subcore_barrier()

    def group_start(s):
        return pl.multiple_of(lax.min((g0 + s) * GROUP, LAST_START), 8)

    def gather_copy(s, b, k):
        off = pl.multiple_of(group_start(s) - block_start, 8)
        return pltpu.make_async_copy(
            table_sh.at[idx_v.at[pl.ds(off + k * CHUNK, CHUNK)]],
            rows_v.at[b, pl.ds(k * CHUNK, CHUNK)],
            sem_g[b],
        )

    def write_copy(s, b):
        return pltpu.make_async_copy(
            rows_v.at[b], out_hbm.at[pl.ds(group_start(s), GROUP)], sem_w[b])

    for s in range(MAX_G + 1):
        b = s % NBUF
        if s < MAX_G:
            @pl.when(s < n_my)
            def _(s=s, b=b):
                if s >= NBUF:
                    write_copy(s - NBUF, b).wait()
                for k in range(GC):
                    gather_copy(s, b, k).start()
        if s >= 1:
            sp, bp = s - 1, (s - 1) % NBUF
            @pl.when(sp < n_my)
            def _(sp=sp, bp=bp):
                for k in range(GC):
                    gather_copy(sp, bp, k).wait()
                write_copy(sp, bp).start()

    for s in range(max(0, MAX_G - NBUF), MAX_G):
        @pl.when(s < n_my)
        def _(s=s):
            write_copy(s, s % NBUF).wait()


def kernel(global_matrix, node_or_bond_graph_indices):
    idx = node_or_bond_graph_indices.astype(jnp.int32)
    return _gather_sc(global_matrix, idx)
